# baseline (speedup 1.0000x reference)
import jax
import jax.numpy as jnp
from jax.experimental import pallas as pl


def _body(w_ref, out_ref):
    out_ref[...] = w_ref[...][None]


def kernel(tokens, W_pos):
    B = tokens.shape[0]
    S = tokens.shape[1]
    D = W_pos.shape[1]
    return pl.pallas_call(
        _body,
        grid=(B,),
        in_specs=[pl.BlockSpec((S, D), lambda b: (0, 0))],
        out_specs=pl.BlockSpec((1, S, D), lambda b: (b, 0, 0)),
        out_shape=jax.ShapeDtypeStruct((B, S, D), W_pos.dtype),
    )(W_pos[:S])
